# baseline (speedup 1.0000x reference)
import functools

import jax
import jax.numpy as jnp
from jax import lax
from jax.experimental import pallas as pl
from jax.experimental.pallas import tpu as pltpu
from jax.experimental.pallas import tpu_sc as plsc

L_SEQ = 200
WORD_DIM = 64
POS_DIM = 64
HIDDEN = 128
LANES = 16
NS = 16

mesh = plsc.VectorSubcoreMesh(
    core_axis_name="c", subcore_axis_name="s", num_cores=2, num_subcores=NS
)


def _m8(x):
    return pl.multiple_of(x, 8)


@functools.partial(
    pl.kernel,
    out_type=[
        jax.ShapeDtypeStruct((L_SEQ, WORD_DIM), jnp.float32),
        jax.ShapeDtypeStruct((1, HIDDEN), jnp.float32),
    ],
    mesh=mesh,
    scratch_types=[
        pltpu.VMEM((LANES,), jnp.int32),
        pltpu.VMEM((LANES, WORD_DIM), jnp.float32),
        pltpu.VMEM((LANES, POS_DIM), jnp.float32),
        pltpu.VMEM((1, HIDDEN), jnp.float32),
        pltpu.VMEM((NS, HIDDEN), jnp.float32),
        pltpu.VMEM_SHARED((NS, HIDDEN), jnp.float32),
        pltpu.SemaphoreType.DMA,
    ],
)
def _encode(idx_hbm, word_hbm, pos_hbm, outw_hbm, hid_hbm,
            idx_v, rows_v, pos_v, psum_v, part_v, parts_s, sem):
    c = lax.axis_index("c")
    t = lax.axis_index("s")

    @pl.when(c == 0)
    def _():
        b1 = _m8(t * 8)
        b2 = _m8(128 + t * 8)
        pltpu.sync_copy(idx_hbm.at[pl.ds(b1, 8)], idx_v.at[pl.ds(0, 8)])
        pltpu.sync_copy(pos_hbm.at[pl.ds(b1, 8)], pos_v.at[pl.ds(0, 8)])

        @pl.when(t < 9)
        def _():
            pltpu.sync_copy(idx_hbm.at[pl.ds(b2, 8)], idx_v.at[pl.ds(8, 8)])
            pltpu.sync_copy(pos_hbm.at[pl.ds(b2, 8)], pos_v.at[pl.ds(8, 8)])

        @pl.when(t >= 9)
        def _():
            zero = jnp.zeros((LANES,), jnp.float32)
            for j in range(8, LANES):
                for cc in range(4):
                    rows_v[j, pl.ds(cc * LANES, LANES)] = zero
                    pos_v[j, pl.ds(cc * LANES, LANES)] = zero

        toks = idx_v[pl.ds(0, LANES)]
        cps = [
            pltpu.async_copy(
                word_hbm.at[pl.ds(toks[j], 1)], rows_v.at[pl.ds(j, 1)], sem
            )
            for j in range(8)
        ]

        @pl.when(t < 9)
        def _():
            cps2 = [
                pltpu.async_copy(
                    word_hbm.at[pl.ds(toks[j], 1)], rows_v.at[pl.ds(j, 1)], sem
                )
                for j in range(8, LANES)
            ]
            for cp in cps2:
                cp.wait()

        for cp in cps:
            cp.wait()

        pltpu.sync_copy(rows_v.at[pl.ds(0, 8)], outw_hbm.at[pl.ds(b1, 8)])

        @pl.when(t < 9)
        def _():
            pltpu.sync_copy(rows_v.at[pl.ds(8, 8)], outw_hbm.at[pl.ds(b2, 8)])

        for cc in range(4):
            aw = rows_v[0, pl.ds(cc * LANES, LANES)]
            ap = pos_v[0, pl.ds(cc * LANES, LANES)]
            for j in range(1, LANES):
                aw = aw + rows_v[j, pl.ds(cc * LANES, LANES)]
                ap = ap + pos_v[j, pl.ds(cc * LANES, LANES)]
            psum_v[0, pl.ds(cc * LANES, LANES)] = aw
            psum_v[0, pl.ds(WORD_DIM + cc * LANES, LANES)] = ap
        pltpu.sync_copy(psum_v, parts_s.at[pl.ds(t, 1)])

    plsc.subcore_barrier()

    @pl.when((c == 0) & (t == 0))
    def _():
        pltpu.sync_copy(parts_s, part_v)
        scale = jnp.float32(1.0 / L_SEQ)
        for cc in range(8):
            tot = part_v[0, pl.ds(cc * LANES, LANES)]
            for j in range(1, NS):
                tot = tot + part_v[j, pl.ds(cc * LANES, LANES)]
            psum_v[0, pl.ds(cc * LANES, LANES)] = tot * scale
        pltpu.sync_copy(psum_v, hid_hbm)


def kernel(inputs, W_word, W_pos):
    outw, hid = _encode(inputs, W_word, W_pos)
    output = jnp.concatenate([outw, W_pos], axis=1)
    return output, hid.reshape(1, 1, HIDDEN)


# speedup vs baseline: 10.7439x; 10.7439x over previous
import functools

import jax
import jax.numpy as jnp
from jax import lax
from jax.experimental import pallas as pl
from jax.experimental.pallas import tpu as pltpu
from jax.experimental.pallas import tpu_sc as plsc

L_SEQ = 200
WORD_DIM = 64
POS_DIM = 64
HIDDEN = 128
LANES = 16
NS = 16
NB = 8

mesh = plsc.VectorSubcoreMesh(
    core_axis_name="c", subcore_axis_name="s", num_cores=2, num_subcores=NS
)


def _m8(x):
    return pl.multiple_of(x, 8)


@functools.partial(
    pl.kernel,
    out_type=[
        jax.ShapeDtypeStruct((L_SEQ, WORD_DIM), jnp.float32),
        jax.ShapeDtypeStruct((1, HIDDEN), jnp.float32),
    ],
    mesh=mesh,
    compiler_params=pltpu.CompilerParams(needs_layout_passes=False),
    scratch_types=[
        pltpu.VMEM((LANES,), jnp.int32),
        pltpu.VMEM((NB, WORD_DIM, 128), jnp.float32),
        pltpu.VMEM((LANES, WORD_DIM), jnp.float32),
        pltpu.VMEM((LANES, POS_DIM), jnp.float32),
        pltpu.VMEM((1, HIDDEN), jnp.float32),
        pltpu.VMEM((NS, HIDDEN), jnp.float32),
        pltpu.VMEM_SHARED((NS, HIDDEN), jnp.float32),
        pltpu.SemaphoreType.DMA,
    ],
)
def _encode(idx_hbm, wordt_hbm, pos_hbm, outw_hbm, hid_hbm,
            idx_v, blk_v, rows_v, pos_v, psum_v, part_v, parts_s, sem):
    c = lax.axis_index("c")
    t = lax.axis_index("s")

    @pl.when(c == 0)
    def _():
        b1 = _m8(t * 8)
        b2 = _m8(128 + t * 8)
        pltpu.sync_copy(idx_hbm.at[pl.ds(b1, 8)], idx_v.at[pl.ds(0, 8)])
        pltpu.sync_copy(pos_hbm.at[pl.ds(b1, 8)], pos_v.at[pl.ds(0, 8)])

        @pl.when(t < 9)
        def _():
            pltpu.sync_copy(idx_hbm.at[pl.ds(b2, 8)], idx_v.at[pl.ds(8, 8)])
            pltpu.sync_copy(pos_hbm.at[pl.ds(b2, 8)], pos_v.at[pl.ds(8, 8)])

        @pl.when(t >= 9)
        def _():
            zero = jnp.zeros((LANES,), jnp.float32)
            for j in range(8, LANES):
                for cc in range(4):
                    rows_v[j, pl.ds(cc * LANES, LANES)] = zero
                    pos_v[j, pl.ds(cc * LANES, LANES)] = zero

        toks = idx_v[pl.ds(0, LANES)]
        rowids = [lax.iota(jnp.int32, LANES) + cc * LANES for cc in range(4)]

        def fire(j, buf):
            base = pl.multiple_of((toks[j] // 128) * 128, 128)
            return pltpu.async_copy(
                wordt_hbm.at[:, pl.ds(base, 128)], blk_v.at[buf], sem
            )

        def extract(j, buf):
            lanecol = jnp.broadcast_to(toks[j] % 128, (LANES,)).astype(jnp.int32)
            for cc in range(4):
                g = plsc.load_gather(blk_v.at[buf], [rowids[cc], lanecol])
                rows_v[j, pl.ds(cc * LANES, LANES)] = g

        cps = [fire(j, j) for j in range(NB)]
        for cp in cps:
            cp.wait()
        for j in range(NB):
            extract(j, j)

        @pl.when(t < 9)
        def _():
            cps2 = [fire(NB + j, j) for j in range(NB)]
            for cp in cps2:
                cp.wait()
            for j in range(NB):
                extract(NB + j, j)

        pltpu.sync_copy(rows_v.at[pl.ds(0, 8)], outw_hbm.at[pl.ds(b1, 8)])

        @pl.when(t < 9)
        def _():
            pltpu.sync_copy(rows_v.at[pl.ds(8, 8)], outw_hbm.at[pl.ds(b2, 8)])

        for cc in range(4):
            aw = rows_v[0, pl.ds(cc * LANES, LANES)]
            ap = pos_v[0, pl.ds(cc * LANES, LANES)]
            for j in range(1, LANES):
                aw = aw + rows_v[j, pl.ds(cc * LANES, LANES)]
                ap = ap + pos_v[j, pl.ds(cc * LANES, LANES)]
            psum_v[0, pl.ds(cc * LANES, LANES)] = aw
            psum_v[0, pl.ds(WORD_DIM + cc * LANES, LANES)] = ap
        pltpu.sync_copy(psum_v, parts_s.at[pl.ds(t, 1)])

    plsc.subcore_barrier()

    @pl.when((c == 0) & (t == 0))
    def _():
        pltpu.sync_copy(parts_s, part_v)
        scale = jnp.float32(1.0 / L_SEQ)
        for cc in range(8):
            tot = part_v[0, pl.ds(cc * LANES, LANES)]
            for j in range(1, NS):
                tot = tot + part_v[j, pl.ds(cc * LANES, LANES)]
            psum_v[0, pl.ds(cc * LANES, LANES)] = tot * scale
        pltpu.sync_copy(psum_v, hid_hbm)


def kernel(inputs, W_word, W_pos):
    outw, hid = _encode(inputs, W_word.T, W_pos)
    output = jnp.concatenate([outw, W_pos], axis=1)
    return output, hid.reshape(1, 1, HIDDEN)


# speedup vs baseline: 12.1408x; 1.1300x over previous
import functools

import jax
import jax.numpy as jnp
from jax import lax
from jax.experimental import pallas as pl
from jax.experimental.pallas import tpu as pltpu
from jax.experimental.pallas import tpu_sc as plsc

L_SEQ = 200
WORD_DIM = 64
POS_DIM = 64
HIDDEN = 128
LANES = 16
NC = 2
NS = 16
NCHUNK = 25

mesh = plsc.VectorSubcoreMesh(
    core_axis_name="c", subcore_axis_name="s", num_cores=NC, num_subcores=NS
)


def _m8(x):
    return pl.multiple_of(x, 8)


@functools.partial(
    pl.kernel,
    out_type=[
        jax.ShapeDtypeStruct((L_SEQ, HIDDEN), jnp.float32),
        jax.ShapeDtypeStruct((NC, HIDDEN), jnp.float32),
    ],
    mesh=mesh,
    compiler_params=pltpu.CompilerParams(needs_layout_passes=False),
    scratch_types=[
        pltpu.VMEM((LANES,), jnp.int32),
        pltpu.VMEM((8, WORD_DIM, 128), jnp.float32),
        pltpu.VMEM((WORD_DIM, L_SEQ), jnp.float32),
        pltpu.VMEM((8, HIDDEN), jnp.float32),
        pltpu.VMEM((1, HIDDEN), jnp.float32),
        pltpu.VMEM((NS, HIDDEN), jnp.float32),
        pltpu.VMEM_SHARED((NS, HIDDEN), jnp.float32),
        pltpu.SemaphoreType.DMA,
    ],
)
def _encode(idx_hbm, wordt_hbm, post_hbm, out_hbm, hid_hbm,
            idx_v, blk_v, post_v, rows_v, psum_v, part_v, parts_s, sem):
    c = lax.axis_index("c")
    s = lax.axis_index("s")
    wid = s * NC + c

    zero = jnp.zeros((LANES,), jnp.float32)
    for cc in range(8):
        psum_v[0, pl.ds(cc * LANES, LANES)] = zero

    @pl.when(wid < NCHUNK)
    def _():
        base = _m8(wid * 8)
        pltpu.sync_copy(idx_hbm.at[pl.ds(base, 8)], idx_v.at[pl.ds(0, 8)])
        toks = idx_v[pl.ds(0, LANES)]
        rowids = [lax.iota(jnp.int32, LANES) + cc * LANES for cc in range(4)]

        cps = [
            pltpu.async_copy(
                wordt_hbm.at[
                    :, pl.ds(pl.multiple_of((toks[j] // 128) * 128, 128), 128)
                ],
                blk_v.at[j],
                sem,
            )
            for j in range(8)
        ]
        cps.append(pltpu.async_copy(post_hbm, post_v, sem))
        for cp in cps:
            cp.wait()

        for j in range(8):
            lanecol = jnp.broadcast_to(toks[j] % 128, (LANES,)).astype(jnp.int32)
            poscol = jnp.broadcast_to(base + j, (LANES,)).astype(jnp.int32)
            for cc in range(4):
                g = plsc.load_gather(blk_v.at[j], [rowids[cc], lanecol])
                p = plsc.load_gather(post_v, [rowids[cc], poscol])
                rows_v[j, pl.ds(cc * LANES, LANES)] = g
                rows_v[j, pl.ds(WORD_DIM + cc * LANES, LANES)] = p

        pltpu.sync_copy(rows_v, out_hbm.at[pl.ds(base, 8)])

        for cc in range(8):
            acc = rows_v[0, pl.ds(cc * LANES, LANES)]
            for j in range(1, 8):
                acc = acc + rows_v[j, pl.ds(cc * LANES, LANES)]
            psum_v[0, pl.ds(cc * LANES, LANES)] = acc

    pltpu.sync_copy(psum_v, parts_s.at[pl.ds(s, 1)])
    plsc.subcore_barrier()

    @pl.when(s == 0)
    def _():
        pltpu.sync_copy(parts_s, part_v)
        scale = jnp.float32(1.0 / L_SEQ)
        for cc in range(8):
            tot = part_v[0, pl.ds(cc * LANES, LANES)]
            for j in range(1, NS):
                tot = tot + part_v[j, pl.ds(cc * LANES, LANES)]
            psum_v[0, pl.ds(cc * LANES, LANES)] = tot * scale
        pltpu.sync_copy(psum_v, hid_hbm.at[pl.ds(c, 1)])


def kernel(inputs, W_word, W_pos):
    out, hid2 = _encode(inputs, W_word.T, W_pos.T)
    return out, (hid2[0] + hid2[1]).reshape(1, 1, HIDDEN)


# speedup vs baseline: 12.8895x; 1.0617x over previous
import functools

import jax
import jax.numpy as jnp
from jax import lax
from jax.experimental import pallas as pl
from jax.experimental.pallas import tpu as pltpu
from jax.experimental.pallas import tpu_sc as plsc

L_SEQ = 200
WORD_DIM = 64
POS_DIM = 64
HIDDEN = 128
LANES = 16
NC = 2
NS = 16
NCHUNK = 25

mesh = plsc.VectorSubcoreMesh(
    core_axis_name="c", subcore_axis_name="s", num_cores=NC, num_subcores=NS
)


def _m8(x):
    return pl.multiple_of(x, 8)


@functools.partial(
    pl.kernel,
    out_type=[
        jax.ShapeDtypeStruct((L_SEQ, HIDDEN), jnp.float32),
        jax.ShapeDtypeStruct((NC, HIDDEN), jnp.float32),
    ],
    mesh=mesh,
    compiler_params=pltpu.CompilerParams(needs_layout_passes=False),
    scratch_types=[
        pltpu.VMEM((LANES,), jnp.int32),
        pltpu.VMEM((8, WORD_DIM, 128), jnp.float32),
        pltpu.VMEM((WORD_DIM, 128), jnp.float32),
        pltpu.VMEM((8, HIDDEN), jnp.float32),
        pltpu.VMEM((1, HIDDEN), jnp.float32),
        pltpu.VMEM((NS, HIDDEN), jnp.float32),
        pltpu.VMEM_SHARED((NS, HIDDEN), jnp.float32),
        pltpu.SemaphoreType.DMA,
    ],
)
def _encode(idx_hbm, wordt_hbm, post_hbm, out_hbm, hid_hbm,
            idx_v, blk_v, post_v, rows_v, psum_v, part_v, parts_s, sem):
    c = lax.axis_index("c")
    s = lax.axis_index("s")
    wid = s * NC + c

    zero = jnp.zeros((LANES,), jnp.float32)
    for cc in range(8):
        psum_v[0, pl.ds(cc * LANES, LANES)] = zero

    @pl.when(wid < NCHUNK)
    def _():
        base = _m8(wid * 8)
        pos_base = pl.multiple_of(jnp.where(wid < 16, 0, 128), 128)
        cp_pos = pltpu.async_copy(
            post_hbm.at[:, pl.ds(pos_base, 128)], post_v, sem
        )
        pltpu.sync_copy(idx_hbm.at[pl.ds(base, 8)], idx_v.at[pl.ds(0, 8)])
        toks = idx_v[pl.ds(0, LANES)]
        rowids = [lax.iota(jnp.int32, LANES) + cc * LANES for cc in range(4)]

        cps = [
            pltpu.async_copy(
                wordt_hbm.at[
                    :, pl.ds(pl.multiple_of((toks[j] // 128) * 128, 128), 128)
                ],
                blk_v.at[j],
                sem,
            )
            for j in range(8)
        ]
        cps.append(cp_pos)
        for cp in cps:
            cp.wait()

        for j in range(8):
            lanecol = jnp.broadcast_to(toks[j] % 128, (LANES,)).astype(jnp.int32)
            poscol = jnp.broadcast_to(base + j - pos_base, (LANES,)).astype(
                jnp.int32
            )
            for cc in range(4):
                g = plsc.load_gather(blk_v.at[j], [rowids[cc], lanecol])
                p = plsc.load_gather(post_v, [rowids[cc], poscol])
                rows_v[j, pl.ds(cc * LANES, LANES)] = g
                rows_v[j, pl.ds(WORD_DIM + cc * LANES, LANES)] = p

        pltpu.sync_copy(rows_v, out_hbm.at[pl.ds(base, 8)])

        for cc in range(8):
            acc = rows_v[0, pl.ds(cc * LANES, LANES)]
            for j in range(1, 8):
                acc = acc + rows_v[j, pl.ds(cc * LANES, LANES)]
            psum_v[0, pl.ds(cc * LANES, LANES)] = acc

    pltpu.sync_copy(psum_v, parts_s.at[pl.ds(s, 1)])
    plsc.subcore_barrier()

    @pl.when(s == 0)
    def _():
        pltpu.sync_copy(parts_s, part_v)
        scale = jnp.float32(1.0 / L_SEQ)
        for cc in range(8):
            tot = part_v[0, pl.ds(cc * LANES, LANES)]
            for j in range(1, NS):
                tot = tot + part_v[j, pl.ds(cc * LANES, LANES)]
            psum_v[0, pl.ds(cc * LANES, LANES)] = tot * scale
        pltpu.sync_copy(psum_v, hid_hbm.at[pl.ds(c, 1)])


def kernel(inputs, W_word, W_pos):
    out, hid2 = _encode(inputs, W_word.T, W_pos.T)
    return out, (hid2[0] + hid2[1]).reshape(1, 1, HIDDEN)


# speedup vs baseline: 12.9832x; 1.0073x over previous
import functools

import jax
import jax.numpy as jnp
from jax import lax
from jax.experimental import pallas as pl
from jax.experimental.pallas import tpu as pltpu
from jax.experimental.pallas import tpu_sc as plsc

L_SEQ = 200
WORD_DIM = 64
POS_DIM = 64
HIDDEN = 128
LANES = 16
NC = 2
NS = 16
NCHUNK = 25

mesh = plsc.VectorSubcoreMesh(
    core_axis_name="c", subcore_axis_name="s", num_cores=NC, num_subcores=NS
)


def _m8(x):
    return pl.multiple_of(x, 8)


@functools.partial(
    pl.kernel,
    out_type=[
        jax.ShapeDtypeStruct((L_SEQ, HIDDEN), jnp.float32),
        jax.ShapeDtypeStruct((NC, HIDDEN), jnp.float32),
    ],
    mesh=mesh,
    compiler_params=pltpu.CompilerParams(needs_layout_passes=False),
    scratch_types=[
        pltpu.VMEM((LANES,), jnp.int32),
        pltpu.VMEM((8, WORD_DIM, 128), jnp.float32),
        pltpu.VMEM((WORD_DIM, 128), jnp.float32),
        pltpu.VMEM((8, HIDDEN), jnp.float32),
        pltpu.VMEM((1, HIDDEN), jnp.float32),
        pltpu.VMEM((NS, HIDDEN), jnp.float32),
        pltpu.VMEM_SHARED((NS, HIDDEN), jnp.float32),
        pltpu.SemaphoreType.DMA,
        pltpu.SemaphoreType.DMA,
    ],
)
def _encode(idx_hbm, wordt_hbm, post_hbm, out_hbm, hid_hbm,
            idx_v, blk_v, post_v, rows_v, psum_v, part_v, parts_s, sem,
            sem_idx):
    c = lax.axis_index("c")
    s = lax.axis_index("s")
    wid = s * NC + c

    zero = jnp.zeros((LANES,), jnp.float32)
    for cc in range(8):
        psum_v[0, pl.ds(cc * LANES, LANES)] = zero

    @pl.when(wid < NCHUNK)
    def _():
        base = _m8(wid * 8)
        cp_idx = pltpu.async_copy(
            idx_hbm.at[pl.ds(base, 8)], idx_v.at[pl.ds(0, 8)], sem_idx
        )
        pos_base = pl.multiple_of(jnp.where(wid < 16, 0, 128), 128)
        cp_pos = pltpu.async_copy(
            post_hbm.at[:, pl.ds(pos_base, 128)], post_v, sem
        )
        cp_idx.wait()
        toks = idx_v[pl.ds(0, LANES)]
        rowids = [lax.iota(jnp.int32, LANES) + cc * LANES for cc in range(4)]

        cps = [
            pltpu.async_copy(
                wordt_hbm.at[
                    :, pl.ds(pl.multiple_of((toks[j] // 128) * 128, 128), 128)
                ],
                blk_v.at[j],
                sem,
            )
            for j in range(8)
        ]
        cps.append(cp_pos)
        for cp in cps:
            cp.wait()

        for j in range(8):
            lanecol = jnp.broadcast_to(toks[j] % 128, (LANES,)).astype(jnp.int32)
            poscol = jnp.broadcast_to(base + j - pos_base, (LANES,)).astype(
                jnp.int32
            )
            for cc in range(4):
                g = plsc.load_gather(blk_v.at[j], [rowids[cc], lanecol])
                p = plsc.load_gather(post_v, [rowids[cc], poscol])
                rows_v[j, pl.ds(cc * LANES, LANES)] = g
                rows_v[j, pl.ds(WORD_DIM + cc * LANES, LANES)] = p

        cp_out = pltpu.async_copy(rows_v, out_hbm.at[pl.ds(base, 8)], sem)

        for cc in range(8):
            acc = rows_v[0, pl.ds(cc * LANES, LANES)]
            for j in range(1, 8):
                acc = acc + rows_v[j, pl.ds(cc * LANES, LANES)]
            psum_v[0, pl.ds(cc * LANES, LANES)] = acc
        cp_out.wait()

    pltpu.sync_copy(psum_v, parts_s.at[pl.ds(s, 1)])
    plsc.subcore_barrier()

    @pl.when(s == 0)
    def _():
        pltpu.sync_copy(parts_s, part_v)
        scale = jnp.float32(1.0 / L_SEQ)
        for cc in range(8):
            tot = part_v[0, pl.ds(cc * LANES, LANES)]
            for j in range(1, NS):
                tot = tot + part_v[j, pl.ds(cc * LANES, LANES)]
            psum_v[0, pl.ds(cc * LANES, LANES)] = tot * scale
        pltpu.sync_copy(psum_v, hid_hbm.at[pl.ds(c, 1)])


def kernel(inputs, W_word, W_pos):
    out, hid2 = _encode(inputs, W_word.T, W_pos.T)
    return out, (hid2[0] + hid2[1]).reshape(1, 1, HIDDEN)


# speedup vs baseline: 12.9994x; 1.0012x over previous
import functools

import jax
import jax.numpy as jnp
from jax import lax
from jax.experimental import pallas as pl
from jax.experimental.pallas import tpu as pltpu
from jax.experimental.pallas import tpu_sc as plsc

L_SEQ = 200
WORD_DIM = 64
POS_DIM = 64
HIDDEN = 128
LANES = 16
NC = 2
NS = 16
NCHUNK = 25

mesh = plsc.VectorSubcoreMesh(
    core_axis_name="c", subcore_axis_name="s", num_cores=NC, num_subcores=NS
)


def _m8(x):
    return pl.multiple_of(x, 8)


@functools.partial(
    pl.kernel,
    out_type=[
        jax.ShapeDtypeStruct((L_SEQ, HIDDEN), jnp.float32),
        jax.ShapeDtypeStruct((NC, HIDDEN), jnp.float32),
    ],
    mesh=mesh,
    compiler_params=pltpu.CompilerParams(
        needs_layout_passes=False,
        disable_bounds_checks=True,
        disable_semaphore_checks=True,
    ),
    scratch_types=[
        pltpu.VMEM((LANES,), jnp.int32),
        pltpu.VMEM((8, WORD_DIM, 128), jnp.float32),
        pltpu.VMEM((WORD_DIM, 128), jnp.float32),
        pltpu.VMEM((8, HIDDEN), jnp.float32),
        pltpu.VMEM((1, HIDDEN), jnp.float32),
        pltpu.VMEM((NS, HIDDEN), jnp.float32),
        pltpu.VMEM_SHARED((NS, HIDDEN), jnp.float32),
        pltpu.SemaphoreType.DMA,
        pltpu.SemaphoreType.DMA,
        pltpu.SemaphoreType.DMA,
    ],
)
def _encode(idx_hbm, wordt_hbm, post_hbm, out_hbm, hid_hbm,
            idx_v, blk_v, post_v, rows_v, psum_v, part_v, parts_s, sem,
            sem_idx, sem_pos):
    c = lax.axis_index("c")
    s = lax.axis_index("s")
    wid = s * NC + c

    zero = jnp.zeros((LANES,), jnp.float32)
    for cc in range(8):
        psum_v[0, pl.ds(cc * LANES, LANES)] = zero

    @pl.when(wid < NCHUNK)
    def _():
        base = _m8(wid * 8)
        cp_idx = pltpu.async_copy(
            idx_hbm.at[pl.ds(base, 8)], idx_v.at[pl.ds(0, 8)], sem_idx
        )
        pos_base = pl.multiple_of(jnp.where(wid < 16, 0, 128), 128)
        cp_pos = pltpu.async_copy(
            post_hbm.at[:, pl.ds(pos_base, 128)], post_v, sem_pos
        )
        cp_idx.wait()
        toks = idx_v[pl.ds(0, LANES)]
        rowids = [lax.iota(jnp.int32, LANES) + cc * LANES for cc in range(4)]

        cps = [
            pltpu.async_copy(
                wordt_hbm.at[
                    :, pl.ds(pl.multiple_of((toks[j] // 128) * 128, 128), 128)
                ],
                blk_v.at[j],
                sem,
            )
            for j in range(8)
        ]
        cp_pos.wait()
        for j in range(8):
            poscol = jnp.broadcast_to(base + j - pos_base, (LANES,)).astype(
                jnp.int32
            )
            for cc in range(4):
                p = plsc.load_gather(post_v, [rowids[cc], poscol])
                rows_v[j, pl.ds(WORD_DIM + cc * LANES, LANES)] = p

        for cp in cps:
            cp.wait()

        for j in range(8):
            lanecol = jnp.broadcast_to(toks[j] % 128, (LANES,)).astype(jnp.int32)
            for cc in range(4):
                g = plsc.load_gather(blk_v.at[j], [rowids[cc], lanecol])
                rows_v[j, pl.ds(cc * LANES, LANES)] = g

        cp_out = pltpu.async_copy(rows_v, out_hbm.at[pl.ds(base, 8)], sem)

        for cc in range(8):
            acc = rows_v[0, pl.ds(cc * LANES, LANES)]
            for j in range(1, 8):
                acc = acc + rows_v[j, pl.ds(cc * LANES, LANES)]
            psum_v[0, pl.ds(cc * LANES, LANES)] = acc
        cp_out.wait()

    pltpu.sync_copy(psum_v, parts_s.at[pl.ds(s, 1)])
    plsc.subcore_barrier()

    @pl.when(s == 0)
    def _():
        pltpu.sync_copy(parts_s, part_v)
        scale = jnp.float32(1.0 / L_SEQ)
        for cc in range(8):
            tot = part_v[0, pl.ds(cc * LANES, LANES)]
            for j in range(1, NS):
                tot = tot + part_v[j, pl.ds(cc * LANES, LANES)]
            psum_v[0, pl.ds(cc * LANES, LANES)] = tot * scale
        pltpu.sync_copy(psum_v, hid_hbm.at[pl.ds(c, 1)])


def kernel(inputs, W_word, W_pos):
    out, hid2 = _encode(inputs, W_word.T, W_pos.T)
    return out, (hid2[0] + hid2[1]).reshape(1, 1, HIDDEN)


# speedup vs baseline: 13.0151x; 1.0012x over previous
import functools

import jax
import jax.numpy as jnp
from jax import lax
from jax.experimental import pallas as pl
from jax.experimental.pallas import tpu as pltpu
from jax.experimental.pallas import tpu_sc as plsc

L_SEQ = 200
WORD_DIM = 64
POS_DIM = 64
HIDDEN = 128
LANES = 16
NC = 2
NS = 16
NCHUNK = 25

mesh = plsc.VectorSubcoreMesh(
    core_axis_name="c", subcore_axis_name="s", num_cores=NC, num_subcores=NS
)


def _m8(x):
    return pl.multiple_of(x, 8)


@functools.partial(
    pl.kernel,
    out_type=[
        jax.ShapeDtypeStruct((L_SEQ, HIDDEN), jnp.float32),
        jax.ShapeDtypeStruct((NC, HIDDEN), jnp.float32),
    ],
    mesh=mesh,
    compiler_params=pltpu.CompilerParams(
        needs_layout_passes=False,
        disable_bounds_checks=True,
        disable_semaphore_checks=True,
        skip_device_barrier=True,
    ),
    scratch_types=[
        pltpu.VMEM((LANES,), jnp.int32),
        pltpu.VMEM((8, WORD_DIM, 128), jnp.float32),
        pltpu.VMEM((WORD_DIM, 128), jnp.float32),
        pltpu.VMEM((8, HIDDEN), jnp.float32),
        pltpu.VMEM((1, HIDDEN), jnp.float32),
        pltpu.VMEM((NS, HIDDEN), jnp.float32),
        pltpu.VMEM_SHARED((NS, HIDDEN), jnp.float32),
        pltpu.SemaphoreType.DMA,
        pltpu.SemaphoreType.DMA,
        pltpu.SemaphoreType.DMA,
    ],
)
def _encode(idx_hbm, wordt_hbm, post_hbm, out_hbm, hid_hbm,
            idx_v, blk_v, post_v, rows_v, psum_v, part_v, parts_s, sem,
            sem_idx, sem_pos):
    c = lax.axis_index("c")
    s = lax.axis_index("s")
    wid = s * NC + c

    zero = jnp.zeros((LANES,), jnp.float32)
    for cc in range(8):
        psum_v[0, pl.ds(cc * LANES, LANES)] = zero

    @pl.when(wid < NCHUNK)
    def _():
        base = _m8(wid * 8)
        cp_idx = pltpu.async_copy(
            idx_hbm.at[pl.ds(base, 8)], idx_v.at[pl.ds(0, 8)], sem_idx
        )
        pos_base = pl.multiple_of(jnp.where(wid < 16, 0, 128), 128)
        cp_pos = pltpu.async_copy(
            post_hbm.at[:, pl.ds(pos_base, 128)], post_v, sem_pos
        )
        cp_idx.wait()
        toks = idx_v[pl.ds(0, LANES)]
        rowids = [lax.iota(jnp.int32, LANES) + cc * LANES for cc in range(4)]

        cps = [
            pltpu.async_copy(
                wordt_hbm.at[
                    :, pl.ds(pl.multiple_of((toks[j] // 128) * 128, 128), 128)
                ],
                blk_v.at[j],
                sem,
            )
            for j in range(8)
        ]
        cp_pos.wait()
        for j in range(8):
            poscol = jnp.broadcast_to(base + j - pos_base, (LANES,)).astype(
                jnp.int32
            )
            for cc in range(4):
                p = plsc.load_gather(post_v, [rowids[cc], poscol])
                rows_v[j, pl.ds(WORD_DIM + cc * LANES, LANES)] = p

        for cp in cps:
            cp.wait()

        for j in range(8):
            lanecol = jnp.broadcast_to(toks[j] % 128, (LANES,)).astype(jnp.int32)
            for cc in range(4):
                g = plsc.load_gather(blk_v.at[j], [rowids[cc], lanecol])
                rows_v[j, pl.ds(cc * LANES, LANES)] = g

        cp_out = pltpu.async_copy(rows_v, out_hbm.at[pl.ds(base, 8)], sem)

        for cc in range(8):
            acc = rows_v[0, pl.ds(cc * LANES, LANES)]
            for j in range(1, 8):
                acc = acc + rows_v[j, pl.ds(cc * LANES, LANES)]
            psum_v[0, pl.ds(cc * LANES, LANES)] = acc
        cp_out.wait()

    pltpu.sync_copy(psum_v, parts_s.at[pl.ds(s, 1)])
    plsc.subcore_barrier()

    @pl.when(s == 0)
    def _():
        pltpu.sync_copy(parts_s, part_v)
        scale = jnp.float32(1.0 / L_SEQ)
        for cc in range(8):
            tot = part_v[0, pl.ds(cc * LANES, LANES)]
            for j in range(1, NS):
                tot = tot + part_v[j, pl.ds(cc * LANES, LANES)]
            psum_v[0, pl.ds(cc * LANES, LANES)] = tot * scale
        pltpu.sync_copy(psum_v, hid_hbm.at[pl.ds(c, 1)])


def kernel(inputs, W_word, W_pos):
    out, hid2 = _encode(inputs, W_word.T, W_pos.T)
    return out, (hid2[0] + hid2[1]).reshape(1, 1, HIDDEN)
